# baseline (speedup 1.0000x reference)
import functools

import jax
import jax.numpy as jnp
from jax import lax
from jax.experimental import pallas as pl
from jax.experimental.pallas import tpu as pltpu
from jax.experimental.pallas import tpu_sc as plsc

_EPS = 1e-06


def _rmsnorm_body(x_ref, w_ref, o_ref):
    x = x_ref[...]
    var = jnp.mean(x * x, axis=-1, keepdims=True)
    o_ref[...] = w_ref[...] * (x * lax.rsqrt(var + _EPS))


def _rmsnorm(x2d, w, block_rows=256):
    rows, hid = x2d.shape
    blk = pl.BlockSpec((block_rows, hid), lambda i: (i, 0))
    return pl.pallas_call(
        _rmsnorm_body,
        grid=(rows // block_rows,),
        in_specs=[blk, pl.BlockSpec((hid,), lambda i: (0,))],
        out_specs=blk,
        out_shape=jax.ShapeDtypeStruct((rows, hid), jnp.float32),
    )(x2d, w)


def _sc_gather(concept_embed, we_flat, normed, ws_off, ns):
    nc_, cdp = concept_embed.shape
    ne = we_flat.shape[0]
    nb = ws_off.shape[0]
    hid = normed.shape[1]
    nw_words = nb // ns

    info = plsc.get_sparse_core_info()
    ncores, nsub = info.num_cores, info.num_subcores
    ntiles = ncores * nsub

    e_per_tile = ne // ntiles
    e_chunk = 128
    e_steps = e_per_tile // e_chunk
    b_per_tile = nb // ntiles
    b_chunk = 4 * ns
    b_steps = b_per_tile // b_chunk
    w_per_chunk = b_chunk // ns
    lanes = info.num_lanes

    mesh = plsc.VectorSubcoreMesh(core_axis_name="c", subcore_axis_name="s")

    @functools.partial(
        pl.kernel,
        mesh=mesh,
        out_type=[
            jax.ShapeDtypeStruct((ne, cdp), jnp.float32),
            jax.ShapeDtypeStruct((nw_words, hid), jnp.float32),
        ],
        scratch_types=[
            pltpu.VMEM((e_chunk,), jnp.int32),
            pltpu.VMEM((e_chunk, cdp), jnp.float32),
            pltpu.VMEM((b_chunk,), jnp.int32),
            pltpu.VMEM((b_chunk, hid), jnp.float32),
            pltpu.VMEM((w_per_chunk, hid), jnp.float32),
            pltpu.SemaphoreType.DMA,
        ],
    )
    def sc_kernel(ce_hbm, we_hbm, normed_hbm, ws_hbm, ents_hbm, bsum_hbm,
                  eidx_v, erow_v, bidx_v, brow_v, bacc_v, sem):
        wid = lax.axis_index("s") * ncores + lax.axis_index("c")

        for ch in range(e_steps):
            base = wid * e_per_tile + ch * e_chunk
            pltpu.sync_copy(we_hbm.at[pl.ds(base, e_chunk)], eidx_v)
            pltpu.async_copy(ce_hbm.at[eidx_v], erow_v, sem).wait()
            pltpu.sync_copy(erow_v, ents_hbm.at[pl.ds(base, e_chunk)])

        for ch in range(b_steps):
            jbase = wid * b_per_tile + ch * b_chunk
            pltpu.sync_copy(ws_hbm.at[pl.ds(jbase, b_chunk)], bidx_v)
            pltpu.async_copy(normed_hbm.at[bidx_v], brow_v, sem).wait()

            def _sum_body(v, carry):
                sl = pl.ds(v * lanes, lanes)
                for w in range(w_per_chunk):
                    acc = brow_v[ns * w, sl]
                    for s in range(1, ns):
                        acc = acc + brow_v[ns * w + s, sl]
                    bacc_v[w, sl] = acc
                return carry

            lax.fori_loop(0, hid // lanes, _sum_body, 0)
            out_row = wid * (b_per_tile // ns) + ch * w_per_chunk
            pltpu.sync_copy(bacc_v, bsum_hbm.at[pl.ds(out_row, w_per_chunk)])

    return sc_kernel(concept_embed, we_flat, normed, ws_off)


def _attention_body(bsum_ref, down_ref, ents_ref, gate_ref, up_ref,
                    sent_ref, out_ref, *, nk, inv_ns):
    bw = bsum_ref.shape[0]
    cdp = ents_ref.shape[1]
    c = lax.dot_general(
        bsum_ref[...], down_ref[...],
        (((1,), (1,)), ((), ())),
        preferred_element_type=jnp.float32) * inv_ns
    x = ents_ref[...]
    g = jnp.dot(x, gate_ref[...], preferred_element_type=jnp.float32)
    u = jnp.dot(x, up_ref[...], preferred_element_type=jnp.float32)
    h = (g * jax.nn.sigmoid(g)) * u
    h3 = h.reshape(bw, nk, h.shape[-1])
    atten = jnp.sum(h3 * c[:, None, :], axis=-1)
    s = sent_ref[...]
    gs = jnp.dot(s, gate_ref[...], preferred_element_type=jnp.float32)
    us = jnp.dot(s, up_ref[...], preferred_element_type=jnp.float32)
    hsent = (gs * jax.nn.sigmoid(gs)) * us
    atten_s = jnp.sum(hsent * c, axis=-1, keepdims=True)
    att = jnp.concatenate([atten, atten_s], axis=1)
    m = jnp.max(att, axis=1, keepdims=True)
    e = jnp.exp(att - m)
    p = e / jnp.sum(e, axis=1, keepdims=True)
    x3 = x.reshape(bw, nk, cdp)
    ao = jnp.sum(p[:, :nk, None] * x3, axis=1) + p[:, nk:] * s
    out_ref[...] = ao


def _attention(bsum, down_w, ents, gate_w, up_w, sent, nk, ns, block_w=256):
    nwords, hid = bsum.shape
    cdp = ents.shape[1]
    inter = gate_w.shape[1]
    grid = (nwords // block_w,)
    return pl.pallas_call(
        functools.partial(_attention_body, nk=nk, inv_ns=1.0 / ns),
        grid=grid,
        in_specs=[
            pl.BlockSpec((block_w, hid), lambda i: (i, 0)),
            pl.BlockSpec((inter, hid), lambda i: (0, 0)),
            pl.BlockSpec((block_w * nk, cdp), lambda i: (i, 0)),
            pl.BlockSpec((cdp, inter), lambda i: (0, 0)),
            pl.BlockSpec((cdp, inter), lambda i: (0, 0)),
            pl.BlockSpec((1, cdp), lambda i: (0, 0)),
        ],
        out_specs=pl.BlockSpec((block_w, cdp), lambda i: (i, 0)),
        out_shape=jax.ShapeDtypeStruct((nwords, cdp), jnp.float32),
    )(bsum, down_w, ents, gate_w, up_w, sent)


def _matmul_body(a_ref, w_ref, o_ref, acc_ref, *, nsteps):
    k = pl.program_id(2)

    @pl.when(k == 0)
    def _():
        acc_ref[...] = jnp.zeros_like(acc_ref)

    acc_ref[...] += jnp.dot(a_ref[...], w_ref[...],
                            preferred_element_type=jnp.float32)

    @pl.when(k == nsteps - 1)
    def _():
        o_ref[...] = acc_ref[...]


def _big_matmul(a, w, bm=1024, bn=1024, bk=1024):
    m, kdim = a.shape
    n = w.shape[1]
    nsteps = kdim // bk
    grid = (m // bm, n // bn, nsteps)
    return pl.pallas_call(
        functools.partial(_matmul_body, nsteps=nsteps),
        grid=grid,
        in_specs=[
            pl.BlockSpec((bm, bk), lambda i, j, k: (i, k)),
            pl.BlockSpec((bk, bn), lambda i, j, k: (k, j)),
        ],
        out_specs=pl.BlockSpec((bm, bn), lambda i, j, k: (i, j)),
        out_shape=jax.ShapeDtypeStruct((m, n), jnp.float32),
        scratch_shapes=[pltpu.VMEM((bm, bn), jnp.float32)],
    )(a, w)


def _epilogue_body(ws_ref, ao_ref, w2_ref, b_ref, z_ref, o_ref, *,
                   ns, block_p, nwrites, nw):
    pblk = pl.program_id(1)
    idx = ws_ref[0]
    jio = lax.broadcasted_iota(jnp.int32, (nwrites, block_p), 0)
    pio = lax.broadcasted_iota(jnp.int32, (nwrites, block_p), 1) \
        + pblk * block_p
    win = jnp.max(jnp.where(idx == pio, jio, -1), axis=0)
    winw = win // ns
    wio = lax.broadcasted_iota(jnp.int32, (block_p, nw), 1)
    onehot = ((winw[:, None] == wio) & (win[:, None] >= 0)) \
        .astype(jnp.float32)
    tmp = jnp.dot(onehot, ao_ref[0], preferred_element_type=jnp.float32)
    z = z_ref[0] + jnp.dot(tmp, w2_ref[...],
                           preferred_element_type=jnp.float32) + b_ref[...]
    o_ref[0] = z * jax.nn.sigmoid(z)


def _epilogue(ws3, attn_out, w2, bias, z, seq, ns, block_p=256):
    bsz, nwrites, _ = ws3.shape
    nw, cdp = attn_out.shape[1], attn_out.shape[2]
    hid = z.shape[-1]
    grid = (bsz, seq // block_p)
    return pl.pallas_call(
        functools.partial(_epilogue_body, ns=ns, block_p=block_p,
                          nwrites=nwrites, nw=nw),
        grid=grid,
        in_specs=[
            pl.BlockSpec((1, nwrites, 1), lambda i, p: (i, 0, 0)),
            pl.BlockSpec((1, nw, cdp), lambda i, p: (i, 0, 0)),
            pl.BlockSpec((cdp, hid), lambda i, p: (0, 0)),
            pl.BlockSpec((1, hid), lambda i, p: (0, 0)),
            pl.BlockSpec((1, block_p, hid), lambda i, p: (i, p, 0)),
        ],
        out_specs=pl.BlockSpec((1, block_p, hid), lambda i, p: (i, p, 0)),
        out_shape=jax.ShapeDtypeStruct((bsz, seq, hid), jnp.float32),
    )(ws3, attn_out, w2, bias, z)


def kernel(output_hidden_states, words_ents_list, words_subtoken_map,
           input_ids, concept_embed, knowledge_sentinel, gate_w, up_w,
           down_w, mlp_w, mlp_b, ln_w):
    bsz, seq, hid = output_hidden_states.shape
    nw, nk = words_ents_list.shape[1], words_ents_list.shape[2]
    ns = words_subtoken_map.shape[2]
    cd = concept_embed.shape[1]
    cdp = 128

    ce_pad = jnp.pad(concept_embed, ((0, 0), (0, cdp - cd)))
    normed = _rmsnorm(output_hidden_states.reshape(bsz * seq, hid), ln_w)

    we_flat = words_ents_list.astype(jnp.int32).reshape(-1)
    ws = words_subtoken_map.astype(jnp.int32)
    boff = (jnp.arange(bsz, dtype=jnp.int32) * seq)[:, None, None]
    ws_off = (ws + boff).reshape(-1)

    z = _big_matmul(normed, mlp_w)
    ents, bsum = _sc_gather(ce_pad, we_flat, normed, ws_off, ns)

    attn_out = _attention(bsum, down_w, ents,
                          jnp.pad(gate_w, ((0, cdp - cd), (0, 0))),
                          jnp.pad(up_w, ((0, cdp - cd), (0, 0))),
                          jnp.pad(knowledge_sentinel,
                                  ((0, 0), (0, cdp - cd))), nk, ns)

    ws3 = ws.reshape(bsz, nw * ns, 1)
    out = _epilogue(ws3, attn_out.reshape(bsz, nw, cdp),
                    jnp.pad(mlp_w[hid:], ((0, cdp - cd), (0, 0))),
                    mlp_b.reshape(1, hid), z.reshape(bsz, seq, hid),
                    seq, ns)
    return out


# speedup vs baseline: 1.1694x; 1.1694x over previous
import functools

import jax
import jax.numpy as jnp
from jax import lax
from jax.experimental import pallas as pl
from jax.experimental.pallas import tpu as pltpu
from jax.experimental.pallas import tpu_sc as plsc

_EPS = 1e-06


def _pad_body(x_ref, o_ref, *, extra):
    x = x_ref[...]
    o_ref[...] = jnp.concatenate(
        [x, jnp.zeros((x.shape[0], extra), x.dtype)], axis=1)


def _pad_lanes(x, cdp, block_rows=5000):
    rows, cd = x.shape
    return pl.pallas_call(
        functools.partial(_pad_body, extra=cdp - cd),
        grid=(rows // block_rows,),
        in_specs=[pl.BlockSpec((block_rows, cd), lambda i: (i, 0))],
        out_specs=pl.BlockSpec((block_rows, cdp), lambda i: (i, 0)),
        out_shape=jax.ShapeDtypeStruct((rows, cdp), jnp.float32),
    )(x)


def _rmsnorm_body(x_ref, w_ref, o_ref):
    x = x_ref[...]
    var = jnp.mean(x * x, axis=-1, keepdims=True)
    o_ref[...] = w_ref[...] * (x * lax.rsqrt(var + _EPS))


def _rmsnorm(x2d, w, block_rows=256):
    rows, hid = x2d.shape
    blk = pl.BlockSpec((block_rows, hid), lambda i: (i, 0))
    return pl.pallas_call(
        _rmsnorm_body,
        grid=(rows // block_rows,),
        in_specs=[blk, pl.BlockSpec((hid,), lambda i: (0,))],
        out_specs=blk,
        out_shape=jax.ShapeDtypeStruct((rows, hid), jnp.float32),
    )(x2d, w)


def _sc_gather(concept_embed, we_flat, normed, ws_off, ns):
    nc_, cdp = concept_embed.shape
    ne = we_flat.shape[0]
    nb = ws_off.shape[0]
    hid = normed.shape[1]
    nw_words = nb // ns

    info = plsc.get_sparse_core_info()
    ncores, nsub = info.num_cores, info.num_subcores
    ntiles = ncores * nsub

    e_per_tile = ne // ntiles
    e_chunk = 128
    e_steps = e_per_tile // e_chunk
    b_per_tile = nb // ntiles
    b_chunk = 4 * ns
    b_steps = b_per_tile // b_chunk
    w_per_chunk = b_chunk // ns
    lanes = info.num_lanes

    mesh = plsc.VectorSubcoreMesh(core_axis_name="c", subcore_axis_name="s")

    @functools.partial(
        pl.kernel,
        mesh=mesh,
        out_type=[
            jax.ShapeDtypeStruct((ne, cdp), jnp.float32),
            jax.ShapeDtypeStruct((nw_words, hid), jnp.float32),
        ],
        scratch_types=[
            pltpu.VMEM((e_chunk,), jnp.int32),
            pltpu.VMEM((e_chunk, cdp), jnp.float32),
            pltpu.VMEM((b_chunk,), jnp.int32),
            pltpu.VMEM((b_chunk, hid), jnp.float32),
            pltpu.VMEM((w_per_chunk, hid), jnp.float32),
            pltpu.SemaphoreType.DMA,
        ],
    )
    def sc_kernel(ce_hbm, we_hbm, normed_hbm, ws_hbm, ents_hbm, bsum_hbm,
                  eidx_v, erow_v, bidx_v, brow_v, bacc_v, sem):
        wid = lax.axis_index("s") * ncores + lax.axis_index("c")

        for ch in range(e_steps):
            base = wid * e_per_tile + ch * e_chunk
            pltpu.sync_copy(we_hbm.at[pl.ds(base, e_chunk)], eidx_v)
            pltpu.async_copy(ce_hbm.at[eidx_v], erow_v, sem).wait()
            pltpu.sync_copy(erow_v, ents_hbm.at[pl.ds(base, e_chunk)])

        for ch in range(b_steps):
            jbase = wid * b_per_tile + ch * b_chunk
            pltpu.sync_copy(ws_hbm.at[pl.ds(jbase, b_chunk)], bidx_v)
            pltpu.async_copy(normed_hbm.at[bidx_v], brow_v, sem).wait()

            def _sum_body(v, carry):
                sl = pl.ds(v * lanes, lanes)
                for w in range(w_per_chunk):
                    acc = brow_v[ns * w, sl]
                    for s in range(1, ns):
                        acc = acc + brow_v[ns * w + s, sl]
                    bacc_v[w, sl] = acc
                return carry

            lax.fori_loop(0, hid // lanes, _sum_body, 0)
            out_row = wid * (b_per_tile // ns) + ch * w_per_chunk
            pltpu.sync_copy(bacc_v, bsum_hbm.at[pl.ds(out_row, w_per_chunk)])

    return sc_kernel(concept_embed, we_flat, normed, ws_off)


def _attention_body(bsum_ref, down_ref, ents_ref, gate_ref, up_ref,
                    sent_ref, out_ref, *, nk, inv_ns):
    bw = bsum_ref.shape[0]
    cdp = ents_ref.shape[1]
    c = lax.dot_general(
        bsum_ref[...], down_ref[...],
        (((1,), (1,)), ((), ())),
        preferred_element_type=jnp.float32) * inv_ns
    x = ents_ref[...]
    g = jnp.dot(x, gate_ref[...], preferred_element_type=jnp.float32)
    u = jnp.dot(x, up_ref[...], preferred_element_type=jnp.float32)
    h = (g * jax.nn.sigmoid(g)) * u
    h3 = h.reshape(bw, nk, h.shape[-1])
    atten = jnp.sum(h3 * c[:, None, :], axis=-1)
    s = sent_ref[...]
    gs = jnp.dot(s, gate_ref[...], preferred_element_type=jnp.float32)
    us = jnp.dot(s, up_ref[...], preferred_element_type=jnp.float32)
    hsent = (gs * jax.nn.sigmoid(gs)) * us
    atten_s = jnp.sum(hsent * c, axis=-1, keepdims=True)
    att = jnp.concatenate([atten, atten_s], axis=1)
    m = jnp.max(att, axis=1, keepdims=True)
    e = jnp.exp(att - m)
    p = e / jnp.sum(e, axis=1, keepdims=True)
    x3 = x.reshape(bw, nk, cdp)
    ao = jnp.sum(p[:, :nk, None] * x3, axis=1) + p[:, nk:] * s
    out_ref[...] = ao


def _attention(bsum, down_w, ents, gate_w, up_w, sent, nk, ns, block_w=256):
    nwords, hid = bsum.shape
    cdp = ents.shape[1]
    inter = gate_w.shape[1]
    grid = (nwords // block_w,)
    return pl.pallas_call(
        functools.partial(_attention_body, nk=nk, inv_ns=1.0 / ns),
        grid=grid,
        in_specs=[
            pl.BlockSpec((block_w, hid), lambda i: (i, 0)),
            pl.BlockSpec((inter, hid), lambda i: (0, 0)),
            pl.BlockSpec((block_w * nk, cdp), lambda i: (i, 0)),
            pl.BlockSpec((cdp, inter), lambda i: (0, 0)),
            pl.BlockSpec((cdp, inter), lambda i: (0, 0)),
            pl.BlockSpec((1, cdp), lambda i: (0, 0)),
        ],
        out_specs=pl.BlockSpec((block_w, cdp), lambda i: (i, 0)),
        out_shape=jax.ShapeDtypeStruct((nwords, cdp), jnp.float32),
    )(bsum, down_w, ents, gate_w, up_w, sent)


def _matmul_body(a_ref, w_ref, o_ref, acc_ref, *, nsteps):
    k = pl.program_id(2)

    @pl.when(k == 0)
    def _():
        acc_ref[...] = jnp.zeros_like(acc_ref)

    acc_ref[...] += jnp.dot(a_ref[...], w_ref[...],
                            preferred_element_type=jnp.float32)

    @pl.when(k == nsteps - 1)
    def _():
        o_ref[...] = acc_ref[...]


def _big_matmul(a, w, bm=1024, bn=1024, bk=1024):
    m, kdim = a.shape
    n = w.shape[1]
    nsteps = kdim // bk
    grid = (m // bm, n // bn, nsteps)
    return pl.pallas_call(
        functools.partial(_matmul_body, nsteps=nsteps),
        grid=grid,
        in_specs=[
            pl.BlockSpec((bm, bk), lambda i, j, k: (i, k)),
            pl.BlockSpec((bk, bn), lambda i, j, k: (k, j)),
        ],
        out_specs=pl.BlockSpec((bm, bn), lambda i, j, k: (i, j)),
        out_shape=jax.ShapeDtypeStruct((m, n), jnp.float32),
        scratch_shapes=[pltpu.VMEM((bm, bn), jnp.float32)],
    )(a, w)


def _epilogue_body(ws_ref, ao_ref, w2_ref, b_ref, z_ref, o_ref, *,
                   ns, block_p, nwrites, nw):
    pblk = pl.program_id(1)
    idx = ws_ref[0]
    jio = lax.broadcasted_iota(jnp.int32, (nwrites, block_p), 0)
    pio = lax.broadcasted_iota(jnp.int32, (nwrites, block_p), 1) \
        + pblk * block_p
    win = jnp.max(jnp.where(idx == pio, jio, -1), axis=0)
    winw = win // ns
    wio = lax.broadcasted_iota(jnp.int32, (block_p, nw), 1)
    onehot = ((winw[:, None] == wio) & (win[:, None] >= 0)) \
        .astype(jnp.float32)
    tmp = jnp.dot(onehot, ao_ref[0], preferred_element_type=jnp.float32)
    z = z_ref[0] + jnp.dot(tmp, w2_ref[...],
                           preferred_element_type=jnp.float32) + b_ref[...]
    o_ref[0] = z * jax.nn.sigmoid(z)


def _epilogue(ws3, attn_out, w2, bias, z, seq, ns, block_p=256):
    bsz, nwrites, _ = ws3.shape
    nw, cdp = attn_out.shape[1], attn_out.shape[2]
    hid = z.shape[-1]
    grid = (bsz, seq // block_p)
    return pl.pallas_call(
        functools.partial(_epilogue_body, ns=ns, block_p=block_p,
                          nwrites=nwrites, nw=nw),
        grid=grid,
        in_specs=[
            pl.BlockSpec((1, nwrites, 1), lambda i, p: (i, 0, 0)),
            pl.BlockSpec((1, nw, cdp), lambda i, p: (i, 0, 0)),
            pl.BlockSpec((cdp, hid), lambda i, p: (0, 0)),
            pl.BlockSpec((1, hid), lambda i, p: (0, 0)),
            pl.BlockSpec((1, block_p, hid), lambda i, p: (i, p, 0)),
        ],
        out_specs=pl.BlockSpec((1, block_p, hid), lambda i, p: (i, p, 0)),
        out_shape=jax.ShapeDtypeStruct((bsz, seq, hid), jnp.float32),
    )(ws3, attn_out, w2, bias, z)


def kernel(output_hidden_states, words_ents_list, words_subtoken_map,
           input_ids, concept_embed, knowledge_sentinel, gate_w, up_w,
           down_w, mlp_w, mlp_b, ln_w):
    bsz, seq, hid = output_hidden_states.shape
    nw, nk = words_ents_list.shape[1], words_ents_list.shape[2]
    ns = words_subtoken_map.shape[2]
    cd = concept_embed.shape[1]
    cdp = 128

    ce_pad = _pad_lanes(concept_embed, cdp)
    normed = _rmsnorm(output_hidden_states.reshape(bsz * seq, hid), ln_w)

    we_flat = words_ents_list.astype(jnp.int32).reshape(-1)
    ws = words_subtoken_map.astype(jnp.int32)
    boff = (jnp.arange(bsz, dtype=jnp.int32) * seq)[:, None, None]
    ws_off = (ws + boff).reshape(-1)

    z = _big_matmul(normed, mlp_w)
    ents, bsum = _sc_gather(ce_pad, we_flat, normed, ws_off, ns)

    attn_out = _attention(bsum, down_w, ents,
                          jnp.pad(gate_w, ((0, cdp - cd), (0, 0))),
                          jnp.pad(up_w, ((0, cdp - cd), (0, 0))),
                          jnp.pad(knowledge_sentinel,
                                  ((0, 0), (0, cdp - cd))), nk, ns)

    ws3 = ws.reshape(bsz, nw * ns, 1)
    out = _epilogue(ws3, attn_out.reshape(bsz, nw, cdp),
                    jnp.pad(mlp_w[hid:], ((0, cdp - cd), (0, 0))),
                    mlp_b.reshape(1, hid), z.reshape(bsz, seq, hid),
                    seq, ns)
    return out


# speedup vs baseline: 1.1777x; 1.0071x over previous
import functools

import jax
import jax.numpy as jnp
from jax import lax
from jax.experimental import pallas as pl
from jax.experimental.pallas import tpu as pltpu
from jax.experimental.pallas import tpu_sc as plsc

_EPS = 1e-06


def _pad_body(x_ref, o_ref, *, extra):
    x = x_ref[...]
    o_ref[...] = jnp.concatenate(
        [x, jnp.zeros((x.shape[0], extra), x.dtype)], axis=1)


def _pad_lanes(x, cdp, block_rows=5000):
    rows, cd = x.shape
    return pl.pallas_call(
        functools.partial(_pad_body, extra=cdp - cd),
        grid=(rows // block_rows,),
        in_specs=[pl.BlockSpec((block_rows, cd), lambda i: (i, 0))],
        out_specs=pl.BlockSpec((block_rows, cdp), lambda i: (i, 0)),
        out_shape=jax.ShapeDtypeStruct((rows, cdp), jnp.float32),
    )(x)


def _rmsnorm_body(x_ref, w_ref, o_ref):
    x = x_ref[...]
    var = jnp.mean(x * x, axis=-1, keepdims=True)
    o_ref[...] = w_ref[...] * (x * lax.rsqrt(var + _EPS))


def _rmsnorm(x2d, w, block_rows=256):
    rows, hid = x2d.shape
    blk = pl.BlockSpec((block_rows, hid), lambda i: (i, 0))
    return pl.pallas_call(
        _rmsnorm_body,
        grid=(rows // block_rows,),
        in_specs=[blk, pl.BlockSpec((hid,), lambda i: (0,))],
        out_specs=blk,
        out_shape=jax.ShapeDtypeStruct((rows, hid), jnp.float32),
    )(x2d, w)


def _sc_gather(concept_embed, we_flat, normed, ws_off, ns):
    nc_, cdp = concept_embed.shape
    ne = we_flat.shape[0]
    nb = ws_off.shape[0]
    hid = normed.shape[1]
    nw_words = nb // ns

    info = plsc.get_sparse_core_info()
    ncores, nsub = info.num_cores, info.num_subcores
    ntiles = ncores * nsub

    e_per_tile = ne // ntiles
    e_chunk = 128
    e_steps = e_per_tile // e_chunk
    b_per_tile = nb // ntiles
    b_chunk = 4 * ns
    b_steps = b_per_tile // b_chunk
    w_per_chunk = b_chunk // ns
    lanes = info.num_lanes

    mesh = plsc.VectorSubcoreMesh(core_axis_name="c", subcore_axis_name="s")

    @functools.partial(
        pl.kernel,
        mesh=mesh,
        out_type=[
            jax.ShapeDtypeStruct((ne, cdp), jnp.float32),
            jax.ShapeDtypeStruct((nw_words, hid), jnp.float32),
        ],
        scratch_types=[
            pltpu.VMEM((e_chunk,), jnp.int32),
            pltpu.VMEM((e_chunk, cdp), jnp.float32),
            pltpu.VMEM((b_chunk,), jnp.int32),
            pltpu.VMEM((b_chunk, hid), jnp.float32),
            pltpu.VMEM((w_per_chunk, hid), jnp.float32),
            pltpu.SemaphoreType.DMA,
        ],
    )
    def sc_kernel(ce_hbm, we_hbm, normed_hbm, ws_hbm, ents_hbm, bsum_hbm,
                  eidx_v, erow_v, bidx_v, brow_v, bacc_v, sem):
        wid = lax.axis_index("s") * ncores + lax.axis_index("c")

        for ch in range(e_steps):
            base = wid * e_per_tile + ch * e_chunk
            pltpu.sync_copy(we_hbm.at[pl.ds(base, e_chunk)], eidx_v)
            pltpu.async_copy(ce_hbm.at[eidx_v], erow_v, sem).wait()
            pltpu.sync_copy(erow_v, ents_hbm.at[pl.ds(base, e_chunk)])

        for ch in range(b_steps):
            jbase = wid * b_per_tile + ch * b_chunk
            pltpu.sync_copy(ws_hbm.at[pl.ds(jbase, b_chunk)], bidx_v)
            pltpu.async_copy(normed_hbm.at[bidx_v], brow_v, sem).wait()

            def _sum_body(v, carry):
                sl = pl.ds(v * lanes, lanes)
                for w in range(w_per_chunk):
                    acc = brow_v[ns * w, sl]
                    for s in range(1, ns):
                        acc = acc + brow_v[ns * w + s, sl]
                    bacc_v[w, sl] = acc
                return carry

            lax.fori_loop(0, hid // lanes, _sum_body, 0)
            out_row = wid * (b_per_tile // ns) + ch * w_per_chunk
            pltpu.sync_copy(bacc_v, bsum_hbm.at[pl.ds(out_row, w_per_chunk)])

    return sc_kernel(concept_embed, we_flat, normed, ws_off)


def _attention_body(bsum_ref, down_ref, ents_ref, gate_ref, up_ref,
                    sent_ref, out_ref, *, nk, inv_ns):
    bw = bsum_ref.shape[0]
    cdp = ents_ref.shape[1]
    c = lax.dot_general(
        bsum_ref[...], down_ref[...],
        (((1,), (1,)), ((), ())),
        preferred_element_type=jnp.float32) * inv_ns
    x = ents_ref[...]
    g = jnp.dot(x, gate_ref[...], preferred_element_type=jnp.float32)
    u = jnp.dot(x, up_ref[...], preferred_element_type=jnp.float32)
    h = (g * jax.nn.sigmoid(g)) * u
    h3 = h.reshape(bw, nk, h.shape[-1])
    atten = jnp.sum(h3 * c[:, None, :], axis=-1)
    s = sent_ref[...]
    gs = jnp.dot(s, gate_ref[...], preferred_element_type=jnp.float32)
    us = jnp.dot(s, up_ref[...], preferred_element_type=jnp.float32)
    hsent = (gs * jax.nn.sigmoid(gs)) * us
    atten_s = jnp.sum(hsent * c, axis=-1, keepdims=True)
    att = jnp.concatenate([atten, atten_s], axis=1)
    m = jnp.max(att, axis=1, keepdims=True)
    e = jnp.exp(att - m)
    p = e / jnp.sum(e, axis=1, keepdims=True)
    x3 = x.reshape(bw, nk, cdp)
    ao = jnp.sum(p[:, :nk, None] * x3, axis=1) + p[:, nk:] * s
    out_ref[...] = ao


def _attention(bsum, down_w, ents, gate_w, up_w, sent, nk, ns, block_w=256):
    nwords, hid = bsum.shape
    cdp = ents.shape[1]
    inter = gate_w.shape[1]
    grid = (nwords // block_w,)
    return pl.pallas_call(
        functools.partial(_attention_body, nk=nk, inv_ns=1.0 / ns),
        grid=grid,
        in_specs=[
            pl.BlockSpec((block_w, hid), lambda i: (i, 0)),
            pl.BlockSpec((inter, hid), lambda i: (0, 0)),
            pl.BlockSpec((block_w * nk, cdp), lambda i: (i, 0)),
            pl.BlockSpec((cdp, inter), lambda i: (0, 0)),
            pl.BlockSpec((cdp, inter), lambda i: (0, 0)),
            pl.BlockSpec((1, cdp), lambda i: (0, 0)),
        ],
        out_specs=pl.BlockSpec((block_w, cdp), lambda i: (i, 0)),
        out_shape=jax.ShapeDtypeStruct((nwords, cdp), jnp.float32),
    )(bsum, down_w, ents, gate_w, up_w, sent)


def _matmul_body(a_ref, w_ref, o_ref, acc_ref, *, nsteps):
    k = pl.program_id(2)

    @pl.when(k == 0)
    def _():
        acc_ref[...] = jnp.zeros_like(acc_ref)

    acc_ref[...] += jnp.dot(a_ref[...], w_ref[...],
                            preferred_element_type=jnp.float32)

    @pl.when(k == nsteps - 1)
    def _():
        o_ref[...] = acc_ref[...]


def _big_matmul(a, w, bm=1024, bn=1024, bk=1024):
    m, kdim = a.shape
    n = w.shape[1]
    nsteps = kdim // bk
    grid = (m // bm, n // bn, nsteps)
    return pl.pallas_call(
        functools.partial(_matmul_body, nsteps=nsteps),
        grid=grid,
        in_specs=[
            pl.BlockSpec((bm, bk), lambda i, j, k: (i, k)),
            pl.BlockSpec((bk, bn), lambda i, j, k: (k, j)),
        ],
        out_specs=pl.BlockSpec((bm, bn), lambda i, j, k: (i, j)),
        out_shape=jax.ShapeDtypeStruct((m, n), jnp.float32),
        scratch_shapes=[pltpu.VMEM((bm, bn), jnp.float32)],
    )(a, w)


def _epilogue_body(ws_ref, ao_ref, w2_ref, b_ref, z_ref, o_ref, *,
                   ns, block_p, nwrites, nw):
    pblk = pl.program_id(1)
    idx = ws_ref[0]
    jio = lax.broadcasted_iota(jnp.int32, (nwrites, block_p), 0)
    pio = lax.broadcasted_iota(jnp.int32, (nwrites, block_p), 1) \
        + pblk * block_p
    win = jnp.max(jnp.where(idx == pio, jio, -1), axis=0)
    winw = win // ns
    wio = lax.broadcasted_iota(jnp.int32, (block_p, nw), 1)
    onehot = ((winw[:, None] == wio) & (win[:, None] >= 0)) \
        .astype(jnp.float32)
    tmp = jnp.dot(onehot, ao_ref[0], preferred_element_type=jnp.float32)
    z = z_ref[0] + jnp.dot(tmp, w2_ref[...],
                           preferred_element_type=jnp.float32) + b_ref[...]
    o_ref[0] = z * jax.nn.sigmoid(z)


def _epilogue(ws3, attn_out, w2, bias, z, seq, ns, block_p=512):
    bsz, nwrites, _ = ws3.shape
    nw, cdp = attn_out.shape[1], attn_out.shape[2]
    hid = z.shape[-1]
    grid = (bsz, seq // block_p)
    return pl.pallas_call(
        functools.partial(_epilogue_body, ns=ns, block_p=block_p,
                          nwrites=nwrites, nw=nw),
        grid=grid,
        in_specs=[
            pl.BlockSpec((1, nwrites, 1), lambda i, p: (i, 0, 0)),
            pl.BlockSpec((1, nw, cdp), lambda i, p: (i, 0, 0)),
            pl.BlockSpec((cdp, hid), lambda i, p: (0, 0)),
            pl.BlockSpec((1, hid), lambda i, p: (0, 0)),
            pl.BlockSpec((1, block_p, hid), lambda i, p: (i, p, 0)),
        ],
        out_specs=pl.BlockSpec((1, block_p, hid), lambda i, p: (i, p, 0)),
        out_shape=jax.ShapeDtypeStruct((bsz, seq, hid), jnp.float32),
    )(ws3, attn_out, w2, bias, z)


def kernel(output_hidden_states, words_ents_list, words_subtoken_map,
           input_ids, concept_embed, knowledge_sentinel, gate_w, up_w,
           down_w, mlp_w, mlp_b, ln_w):
    bsz, seq, hid = output_hidden_states.shape
    nw, nk = words_ents_list.shape[1], words_ents_list.shape[2]
    ns = words_subtoken_map.shape[2]
    cd = concept_embed.shape[1]
    cdp = 128

    ce_pad = _pad_lanes(concept_embed, cdp)
    normed = _rmsnorm(output_hidden_states.reshape(bsz * seq, hid), ln_w)

    we_flat = words_ents_list.astype(jnp.int32).reshape(-1)
    ws = words_subtoken_map.astype(jnp.int32)
    boff = (jnp.arange(bsz, dtype=jnp.int32) * seq)[:, None, None]
    ws_off = (ws + boff).reshape(-1)

    z = _big_matmul(normed, mlp_w)
    ents, bsum = _sc_gather(ce_pad, we_flat, normed, ws_off, ns)

    attn_out = _attention(bsum, down_w, ents,
                          jnp.pad(gate_w, ((0, cdp - cd), (0, 0))),
                          jnp.pad(up_w, ((0, cdp - cd), (0, 0))),
                          jnp.pad(knowledge_sentinel,
                                  ((0, 0), (0, cdp - cd))), nk, ns)

    ws3 = ws.reshape(bsz, nw * ns, 1)
    out = _epilogue(ws3, attn_out.reshape(bsz, nw, cdp),
                    jnp.pad(mlp_w[hid:], ((0, cdp - cd), (0, 0))),
                    mlp_b.reshape(1, hid), z.reshape(bsz, seq, hid),
                    seq, ns)
    return out
